# baseline (speedup 1.0000x reference)
import functools

import jax
import jax.numpy as jnp
from jax import lax
from jax.experimental import pallas as pl
from jax.experimental.pallas import tpu as pltpu
from jax.experimental.pallas import tpu_sc as plsc

_CHUNK = 128
_NC = 2
_NS = 16
_NW = _NC * _NS
_WBC = 12800
_SPB = 4


@functools.lru_cache(maxsize=None)
def _make_gather(n_total, hidden, batch, k):
    per_w = n_total // _NW
    n_chunks = per_w // _CHUNK
    n_groups = n_chunks // k
    n_pairs = n_groups // 2
    cpp = batch // _CHUNK
    hpp = cpp // 2
    rpp = batch // 2
    assert n_total == per_w * _NW
    assert n_chunks * _CHUNK == per_w
    assert n_groups * k == n_chunks and n_pairs * 2 == n_groups
    mesh = plsc.VectorSubcoreMesh(core_axis_name="c", subcore_axis_name="s")

    @functools.partial(
        pl.kernel,
        out_type=jax.ShapeDtypeStruct((n_total // 2, 2 * hidden), jnp.float32),
        mesh=mesh,
        scratch_types=[
            pltpu.VMEM((n_chunks, _CHUNK), jnp.int32),
            pltpu.VMEM((2 * k, _CHUNK, hidden), jnp.float32),
            pltpu.SemaphoreType.DMA,
            pltpu.SemaphoreType.DMA,
            pltpu.SemaphoreType.DMA,
            pltpu.SemaphoreType.DMA,
        ],
        compiler_params=pltpu.CompilerParams(use_tc_tiling_on_sc=False),
    )
    def gather_kernel(idx_hbm, table_hbm, out_hbm, idx_v, rows_v,
                      gsem_a, gsem_b, osem_a, osem_b):
        wid = lax.axis_index("s") * _NC + lax.axis_index("c")
        pltpu.sync_copy(idx_hbm.at[pl.ds(wid * n_chunks, n_chunks)], idx_v)
        gsems = (gsem_a, gsem_b)
        osems = (osem_a, osem_b)

        def out_slice(g, b):
            cg = wid * n_chunks + g * k + b
            s = cg // cpp
            t = cg - s * cpp
            row = s * rpp + (t % hpp) * _CHUNK
            col = (t // hpp) * hidden
            return out_hbm.at[pl.ds(row, _CHUNK), pl.ds(col, hidden)]

        def fire_gathers(g, half):
            for b in range(k):
                pltpu.async_copy(
                    table_hbm.at[idx_v.at[g * k + b]],
                    rows_v.at[half * k + b],
                    gsems[half],
                )

        def drain_gathers(g, half):
            for b in range(k):
                pltpu.make_async_copy(
                    table_hbm.at[idx_v.at[g * k + b]],
                    rows_v.at[half * k + b],
                    gsems[half],
                ).wait()

        def fire_outs(g, half):
            for b in range(k):
                pltpu.async_copy(
                    rows_v.at[half * k + b],
                    out_slice(g, b),
                    osems[half],
                )

        def drain_outs(g, half):
            for b in range(k):
                pltpu.make_async_copy(
                    rows_v.at[half * k + b],
                    out_slice(g, b),
                    osems[half],
                ).wait()

        fire_gathers(0, 0)

        def pair_body(p, carry):
            ga = 2 * p
            gb = ga + 1

            @pl.when(p > 0)
            def _():
                drain_outs(ga - 1, 1)

            fire_gathers(gb, 1)
            drain_gathers(ga, 0)
            fire_outs(ga, 0)
            drain_outs(ga, 0)

            @pl.when(p + 1 < n_pairs)
            def _():
                fire_gathers(ga + 2, 0)

            drain_gathers(gb, 1)
            fire_outs(gb, 1)
            return carry

        lax.fori_loop(0, n_pairs, pair_body, 0)
        drain_outs(n_groups - 1, 1)

    return gather_kernel


def _w_repack_body(x_ref, o_ref):
    x = x_ref[...]
    half = _WBC // 2
    stacked = jnp.concatenate([x[:, :half], x[:, half:]], axis=0)
    o_ref[...] = jnp.transpose(stacked)


def _out_repack_body(x_ref, o_ref):
    z = jnp.transpose(x_ref[...])
    for p in range(_SPB):
        zp = z[:, p * 2048:(p + 1) * 2048]
        o_ref[p] = jnp.concatenate([zp[:64], zp[64:]], axis=1)


def _out_repack_tail_body(x_ref, carry_ref, o_ref):
    del carry_ref
    _out_repack_body(x_ref, o_ref)


def kernel(indices, W):
    B, S = indices.shape
    n_total = B * S
    hidden = W.shape[1]
    n_rows = W.shape[0]
    n_blocks = (n_rows + _WBC - 1) // _WBC

    wt = jnp.transpose(W)
    w_flat = pl.pallas_call(
        _w_repack_body,
        grid=(n_blocks,),
        in_specs=[pl.BlockSpec((hidden, _WBC), lambda i: (0, i))],
        out_specs=pl.BlockSpec((_WBC // 2, 2 * hidden), lambda i: (i, 0)),
        out_shape=jax.ShapeDtypeStruct((n_blocks * _WBC // 2, 2 * hidden),
                                       jnp.float32),
    )(wt)
    w_lin = w_flat.reshape(n_blocks * _WBC, hidden)

    flat = jnp.transpose(indices.astype(jnp.int32)).reshape(-1)
    blk = flat // _WBC
    m = flat - blk * _WBC
    g = blk * _WBC + 2 * (m % (_WBC // 2)) + m // (_WBC // 2)
    half_tok = n_total // 2
    g2 = g.reshape(2, half_tok // _CHUNK, _CHUNK)
    gather = _make_gather(half_tok, hidden, B, 5)
    staged_a = gather(g2[0], w_lin)
    staged_b = gather(g2[1], w_lin)

    half_grid = S // 2 // _SPB
    plane_spec = pl.BlockSpec((_SPB * B // 2, 2 * hidden), lambda i: (i, 0))
    out3a = pl.pallas_call(
        _out_repack_body,
        grid=(half_grid,),
        in_specs=[plane_spec],
        out_specs=pl.BlockSpec((_SPB, hidden, B), lambda i: (i, 0, 0)),
        out_shape=jax.ShapeDtypeStruct((S, hidden, B), jnp.float32),
    )(staged_a)
    out3 = pl.pallas_call(
        _out_repack_tail_body,
        grid=(half_grid,),
        in_specs=[
            plane_spec,
            pl.BlockSpec(memory_space=pltpu.MemorySpace.HBM),
        ],
        out_specs=pl.BlockSpec((_SPB, hidden, B),
                               lambda i: (i + S // 2 // _SPB, 0, 0)),
        out_shape=jax.ShapeDtypeStruct((S, hidden, B), jnp.float32),
        input_output_aliases={1: 0},
    )(staged_b, out3a)
    return jnp.transpose(out3, (2, 0, 1))


# speedup vs baseline: 1.0303x; 1.0303x over previous
import functools

import jax
import jax.numpy as jnp
from jax import lax
from jax.experimental import pallas as pl
from jax.experimental.pallas import tpu as pltpu
from jax.experimental.pallas import tpu_sc as plsc

_CHUNK = 128
_NC = 2
_NS = 16
_NW = _NC * _NS
_WBC = 25600
_SPB = 8


@functools.lru_cache(maxsize=None)
def _make_gather(n_total, hidden, batch, k):
    per_w = n_total // _NW
    n_chunks = per_w // _CHUNK
    n_groups = n_chunks // k
    n_pairs = n_groups // 2
    cpp = batch // _CHUNK
    hpp = cpp // 2
    rpp = batch // 2
    assert n_total == per_w * _NW
    assert n_chunks * _CHUNK == per_w
    assert n_groups * k == n_chunks and n_pairs * 2 == n_groups
    mesh = plsc.VectorSubcoreMesh(core_axis_name="c", subcore_axis_name="s")

    @functools.partial(
        pl.kernel,
        out_type=jax.ShapeDtypeStruct((n_total // 2, 2 * hidden), jnp.float32),
        mesh=mesh,
        scratch_types=[
            pltpu.VMEM((n_chunks, _CHUNK), jnp.int32),
            pltpu.VMEM((2 * k, _CHUNK, hidden), jnp.float32),
            pltpu.SemaphoreType.DMA,
            pltpu.SemaphoreType.DMA,
            pltpu.SemaphoreType.DMA,
            pltpu.SemaphoreType.DMA,
        ],
        compiler_params=pltpu.CompilerParams(use_tc_tiling_on_sc=False),
    )
    def gather_kernel(idx_hbm, table_hbm, out_hbm, idx_v, rows_v,
                      gsem_a, gsem_b, osem_a, osem_b):
        wid = lax.axis_index("s") * _NC + lax.axis_index("c")
        pltpu.sync_copy(idx_hbm.at[pl.ds(wid * n_chunks, n_chunks)], idx_v)
        gsems = (gsem_a, gsem_b)
        osems = (osem_a, osem_b)

        def out_slice(g, b):
            cg = wid * n_chunks + g * k + b
            s = cg // cpp
            t = cg - s * cpp
            row = s * rpp + (t % hpp) * _CHUNK
            col = (t // hpp) * hidden
            return out_hbm.at[pl.ds(row, _CHUNK), pl.ds(col, hidden)]

        def fire_gathers(g, half):
            for b in range(k):
                pltpu.async_copy(
                    table_hbm.at[idx_v.at[g * k + b]],
                    rows_v.at[half * k + b],
                    gsems[half],
                )

        def drain_gathers(g, half):
            for b in range(k):
                pltpu.make_async_copy(
                    table_hbm.at[idx_v.at[g * k + b]],
                    rows_v.at[half * k + b],
                    gsems[half],
                ).wait()

        def fire_outs(g, half):
            for b in range(k):
                pltpu.async_copy(
                    rows_v.at[half * k + b],
                    out_slice(g, b),
                    osems[half],
                )

        def drain_outs(g, half):
            for b in range(k):
                pltpu.make_async_copy(
                    rows_v.at[half * k + b],
                    out_slice(g, b),
                    osems[half],
                ).wait()

        fire_gathers(0, 0)

        def pair_body(p, carry):
            ga = 2 * p
            gb = ga + 1

            @pl.when(p > 0)
            def _():
                drain_outs(ga - 1, 1)

            fire_gathers(gb, 1)
            drain_gathers(ga, 0)
            fire_outs(ga, 0)
            drain_outs(ga, 0)

            @pl.when(p + 1 < n_pairs)
            def _():
                fire_gathers(ga + 2, 0)

            drain_gathers(gb, 1)
            fire_outs(gb, 1)
            return carry

        lax.fori_loop(0, n_pairs, pair_body, 0)
        drain_outs(n_groups - 1, 1)

    return gather_kernel


def _w_repack_body(x_ref, o_ref):
    x = x_ref[...]
    half = _WBC // 2
    stacked = jnp.concatenate([x[:, :half], x[:, half:]], axis=0)
    o_ref[...] = jnp.transpose(stacked)


def _out_repack_body(x_ref, o_ref):
    z = jnp.transpose(x_ref[...])
    for p in range(_SPB):
        zp = z[:, p * 2048:(p + 1) * 2048]
        o_ref[p] = jnp.concatenate([zp[:64], zp[64:]], axis=1)


def kernel(indices, W):
    B, S = indices.shape
    n_total = B * S
    hidden = W.shape[1]
    n_rows = W.shape[0]
    n_blocks = (n_rows + _WBC - 1) // _WBC

    wt = jnp.transpose(W)
    w_flat = pl.pallas_call(
        _w_repack_body,
        grid=(n_blocks,),
        in_specs=[pl.BlockSpec((hidden, _WBC), lambda i: (0, i))],
        out_specs=pl.BlockSpec((_WBC // 2, 2 * hidden), lambda i: (i, 0)),
        out_shape=jax.ShapeDtypeStruct((n_blocks * _WBC // 2, 2 * hidden),
                                       jnp.float32),
    )(wt)
    w_lin = w_flat.reshape(n_blocks * _WBC, hidden)

    flat = jnp.transpose(indices.astype(jnp.int32)).reshape(-1)
    blk = flat // _WBC
    m = flat - blk * _WBC
    g = blk * _WBC + 2 * (m % (_WBC // 2)) + m // (_WBC // 2)
    idx2d = g.reshape(n_total // _CHUNK, _CHUNK)
    staged = _make_gather(n_total, hidden, B, 5)(idx2d, w_lin)

    out3 = pl.pallas_call(
        _out_repack_body,
        grid=(S // _SPB,),
        in_specs=[pl.BlockSpec((_SPB * B // 2, 2 * hidden), lambda i: (i, 0))],
        out_specs=pl.BlockSpec((_SPB, hidden, B), lambda i: (i, 0, 0)),
        out_shape=jax.ShapeDtypeStruct((S, hidden, B), jnp.float32),
    )(staged)
    return jnp.transpose(out3, (2, 0, 1))
